# baseline (speedup 1.0000x reference)
import functools

import jax
import jax.numpy as jnp
from jax import lax
from jax.experimental import pallas as pl
from jax.experimental.pallas import tpu as pltpu
from jax.experimental.pallas import tpu_sc as plsc

_LANES = 16
_NBUF = 15
_LOOKAHEAD = 10


@functools.lru_cache(maxsize=None)
def _build_sc_gather(N, D, B, K):
    info = plsc.get_sparse_core_info()
    NW = info.num_cores * info.num_subcores
    assert D % 8 == 0 and B % ((NW // (D // 8)) * 128) == 0
    nrow = D // 8
    ncolw = NW // nrow
    cs = B // ncolw
    kpad = -(-K // _LANES) * _LANES
    nbuf = min(_NBUF, K)
    look = min(_LOOKAHEAD, nbuf)

    mesh = plsc.VectorSubcoreMesh(core_axis_name="c", subcore_axis_name="s")

    @functools.partial(
        pl.kernel,
        out_type=jax.ShapeDtypeStruct((K, D, B), jnp.float32),
        mesh=mesh,
        scratch_types=[
            pltpu.VMEM((kpad,), jnp.int32),
            pltpu.VMEM((nbuf, 8, cs), jnp.float32),
            pltpu.SemaphoreType.DMA,
            pltpu.SemaphoreType.DMA,
        ],
        compiler_params=pltpu.CompilerParams(
            use_tc_tiling_on_sc=True,
            needs_layout_passes=False,
            disable_bounds_checks=True,
            disable_semaphore_checks=True,
            skip_device_barrier=True,
        ),
    )
    def sc_gather(x_hbm, idx_hbm, out_hbm, idx_v, ring, gsem, wsem):
        wid = lax.axis_index("s") * info.num_cores + lax.axis_index("c")
        r0 = (wid // ncolw) * 8
        c0 = (wid % ncolw) * cs
        pltpu.sync_copy(idx_hbm, idx_v)

        lane_ids = lax.iota(jnp.int32, _LANES)
        vecs = [idx_v[pl.ds(g * _LANES, _LANES)] for g in range(kpad // _LANES)]
        scal = {}

        def idx_scalar(j):
            if j not in scal:
                scal[j] = jnp.sum(
                    jnp.where(lane_ids == (j % _LANES), vecs[j // _LANES], 0)
                )
            return scal[j]

        def gather(j):
            return pltpu.async_copy(
                x_hbm.at[idx_scalar(j), pl.ds(r0, 8), pl.ds(c0, cs)],
                ring.at[j % nbuf],
                gsem,
            )

        def write(j):
            return pltpu.async_copy(
                ring.at[j % nbuf],
                out_hbm.at[j, pl.ds(r0, 8), pl.ds(c0, cs)],
                wsem,
            )

        gc = {}
        wc = {}
        for j in range(look):
            gc[j] = gather(j)
        for j in range(K):
            gc[j].wait()
            wc[j] = write(j)
            nj = j + look
            if nj < K:
                ow = nj - nbuf
                if ow >= 0:
                    wc[ow].wait()
                gc[nj] = gather(nj)
        for j in range(max(0, K - nbuf), K):
            if j >= 0 and wc[j] is not None:
                wc[j].wait()

    return sc_gather


def kernel(x, indices):
    B, N, D = x.shape
    K = indices.shape[0]
    kpad = -(-K // _LANES) * _LANES
    idx_pad = jnp.zeros((kpad,), jnp.int32).at[:K].set(indices.astype(jnp.int32))
    x3 = x.transpose(1, 2, 0)
    out3 = _build_sc_gather(N, D, B, K)(x3, idx_pad)
    return out3.transpose(2, 0, 1)


# speedup vs baseline: 1.0221x; 1.0221x over previous
import functools

import jax
import jax.numpy as jnp
from jax import lax
from jax.experimental import pallas as pl
from jax.experimental.pallas import tpu as pltpu
from jax.experimental.pallas import tpu_sc as plsc

_LANES = 16
_NBUF = 15
_LOOKAHEAD = 10


@functools.lru_cache(maxsize=None)
def _build_sc_gather(N, D, B, K):
    info = plsc.get_sparse_core_info()
    NW = info.num_cores * info.num_subcores
    assert B % (NW * 128) == 0
    nc = B // NW
    kpad = -(-K // _LANES) * _LANES
    nbuf = min(_NBUF, K)
    look = min(_LOOKAHEAD, nbuf)

    mesh = plsc.VectorSubcoreMesh(core_axis_name="c", subcore_axis_name="s")

    @functools.partial(
        pl.kernel,
        out_type=jax.ShapeDtypeStruct((K, D, B), jnp.float32),
        mesh=mesh,
        scratch_types=[
            pltpu.VMEM((kpad,), jnp.int32),
            pltpu.VMEM((nbuf, D, nc), jnp.float32),
            pltpu.SemaphoreType.DMA,
            pltpu.SemaphoreType.DMA,
        ],
        compiler_params=pltpu.CompilerParams(
            use_tc_tiling_on_sc=True,
            needs_layout_passes=False,
            disable_bounds_checks=True,
            disable_semaphore_checks=True,
            skip_device_barrier=True,
        ),
    )
    def sc_gather(x_hbm, idx_hbm, out_hbm, idx_v, ring, gsem, wsem):
        wid = lax.axis_index("s") * info.num_cores + lax.axis_index("c")
        c0 = wid * nc
        pltpu.sync_copy(idx_hbm, idx_v.at[pl.ds(0, K)])

        lane_ids = lax.iota(jnp.int32, _LANES)
        vecs = [idx_v[pl.ds(g * _LANES, _LANES)] for g in range(kpad // _LANES)]
        scal = {}

        def idx_scalar(j):
            if j not in scal:
                scal[j] = jnp.sum(
                    jnp.where(lane_ids == (j % _LANES), vecs[j // _LANES], 0)
                )
            return scal[j]

        def gather(j):
            return pltpu.async_copy(
                x_hbm.at[idx_scalar(j), :, pl.ds(c0, nc)],
                ring.at[j % nbuf],
                gsem,
            )

        def write(j):
            return pltpu.async_copy(
                ring.at[j % nbuf], out_hbm.at[j, :, pl.ds(c0, nc)], wsem
            )

        gc = {}
        wc = {}
        for j in range(look):
            gc[j] = gather(j)
        for j in range(K):
            gc[j].wait()
            wc[j] = write(j)
            nj = j + look
            if nj < K:
                ow = nj - nbuf
                if ow >= 0:
                    wc[ow].wait()
                gc[nj] = gather(nj)
        for j in range(max(0, K - nbuf), K):
            if j >= 0 and wc[j] is not None:
                wc[j].wait()

    return sc_gather


def kernel(x, indices):
    B, N, D = x.shape
    K = indices.shape[0]
    x3 = x.transpose(1, 2, 0)
    out3 = _build_sc_gather(N, D, B, K)(x3, indices.astype(jnp.int32))
    return out3.transpose(2, 0, 1)
